# baseline (speedup 1.0000x reference)
import dataclasses

import jax
import jax.numpy as jnp
from jax import lax
from jax.experimental import pallas as pl
from jax.experimental.pallas import tpu as pltpu
from jax.experimental.pallas import tpu_sc as plsc

N = 10000
D = 128
E = 320000
NC = 2
NS = 16
NW = NC * NS
CHUNK = 128
B_T = 80
EPT = B_T * CHUNK
E_PAD = EPT * NW
W0 = 156
ACC_ROWS = 10112
ACC_BLKS = ACC_ROWS // CHUNK
RPT = ACC_ROWS // NS

_MESH = plsc.VectorSubcoreMesh(
    core_axis_name="c", subcore_axis_name="s", num_cores=NC, num_subcores=NS
)

_CP = pltpu.CompilerParams()
if "needs_layout_passes" in pltpu.CompilerParams.__dataclass_fields__:
    _CP = dataclasses.replace(_CP, needs_layout_passes=False)


def _hist_body(to3_hbm, out_hbm, tbuf_v, cnt_v):
    c = lax.axis_index("c")
    s = lax.axis_index("s")
    wid = c * NS + s

    @pl.loop(0, ACC_ROWS // 16)
    def _(i):
        cnt_v[pl.ds(i * 16, 16)] = jnp.zeros((16,), jnp.float32)

    pltpu.sync_copy(to3_hbm.at[pl.ds(wid * B_T, B_T)], tbuf_v)

    @pl.loop(0, B_T)
    def _(b):
        @pl.loop(0, CHUNK // 16)
        def _(g):
            idx16 = tbuf_v[b, 0, pl.ds(g * 16, 16)]
            plsc.addupdate_scatter(cnt_v, [idx16], jnp.ones((16,), jnp.float32))

    pltpu.sync_copy(cnt_v, out_hbm.at[wid])


_hist = pl.kernel(
    _hist_body,
    out_type=jax.ShapeDtypeStruct((NW, ACC_ROWS), jnp.float32),
    mesh=_MESH,
    scratch_types=[
        pltpu.VMEM((B_T, 1, CHUNK), jnp.int32),
        pltpu.VMEM((ACC_ROWS,), jnp.float32),
    ],
    compiler_params=_CP,
)


def _agg_body(fto_hbm, xs_hbm, out_hbm,
              idx0_v, idx1_v, rows0_v, rows1_v, acc_sh,
              gsem0, gsem1, isem0, isem1, hsem0, hsem1):
    c = lax.axis_index("c")
    s = lax.axis_index("s")

    @pl.loop(0, CHUNK)
    def _(r):
        for g in range(D // 16):
            rows0_v[r, pl.ds(g * 16, 16)] = jnp.zeros((16,), jnp.float32)

    @pl.loop(0, (ACC_BLKS + NS - 1) // NS)
    def _(b):
        blk = b * NS + s

        @pl.when(blk < ACC_BLKS)
        def _():
            pltpu.sync_copy(rows0_v, acc_sh.at[pl.ds(blk * CHUNK, CHUNK)])

    nb = jnp.where(c == 0, W0, 2 * B_T - W0)
    base = jnp.where(c == 0, s * W0, NS * W0 + s * (2 * B_T - W0))
    pltpu.async_copy(fto_hbm.at[base], idx0_v, isem0)
    pltpu.async_copy(fto_hbm.at[base + 1], idx1_v, isem1)
    plsc.subcore_barrier()
    idxs = (idx0_v, idx1_v)
    rows = (rows0_v, rows1_v)
    gsems = (gsem0, gsem1)
    hsems = (hsem0, hsem1)
    isems = (isem0, isem1)

    HC = CHUNK // 2

    def _gather_start(j):
        pltpu.async_copy(
            xs_hbm.at[idxs[j].at[0].at[pl.ds(0, HC)]],
            rows[j].at[pl.ds(0, HC)], gsems[j])
        pltpu.async_copy(
            xs_hbm.at[idxs[j].at[0].at[pl.ds(HC, HC)]],
            rows[j].at[pl.ds(HC, HC)], hsems[j])

    def _gather_wait(j):
        pltpu.make_async_copy(
            xs_hbm.at[idxs[j].at[0].at[pl.ds(0, HC)]],
            rows[j].at[pl.ds(0, HC)], gsems[j]).wait()
        pltpu.make_async_copy(
            xs_hbm.at[idxs[j].at[0].at[pl.ds(HC, HC)]],
            rows[j].at[pl.ds(HC, HC)], hsems[j]).wait()

    pltpu.make_async_copy(fto_hbm.at[base], idx0_v, isem0).wait()
    _gather_start(0)

    @pl.loop(0, nb, step=2)
    def _(b):
        for j in range(2):
            bb = b + j
            j1 = 1 - j

            @pl.when(bb + 1 < nb)
            def _():
                pltpu.make_async_copy(fto_hbm.at[base], idxs[j1], isems[j1]).wait()
                _gather_start(j1)

            _gather_wait(j)
            pltpu.sync_copy(rows[j], acc_sh.at[idxs[j].at[1]], add=True)

            @pl.when(bb + 2 < nb)
            def _():
                pltpu.async_copy(fto_hbm.at[base + bb + 2], idxs[j], isems[j])

    plsc.subcore_barrier()
    pltpu.sync_copy(
        acc_sh.at[pl.ds(s * RPT, RPT)], out_hbm.at[c].at[pl.ds(s * RPT, RPT)]
    )


_agg = pl.kernel(
    _agg_body,
    out_type=jax.ShapeDtypeStruct((NC, ACC_ROWS, D), jnp.float32),
    mesh=_MESH,
    scratch_types=[
        pltpu.VMEM((2, CHUNK), jnp.int32),
        pltpu.VMEM((2, CHUNK), jnp.int32),
        pltpu.VMEM((CHUNK, D), jnp.float32),
        pltpu.VMEM((CHUNK, D), jnp.float32),
        pltpu.VMEM_SHARED((ACC_ROWS, D), jnp.float32),
        pltpu.SemaphoreType.DMA,
        pltpu.SemaphoreType.DMA,
        pltpu.SemaphoreType.DMA,
        pltpu.SemaphoreType.DMA,
        pltpu.SemaphoreType.DMA,
        pltpu.SemaphoreType.DMA,
    ],
)


def _prep_body(degp_ref, x_ref, xs_ref, dinv_ref):
    deg = jnp.sum(degp_ref[...], axis=0, keepdims=True)
    dinv = jnp.where(deg > 0.0, lax.rsqrt(deg), 0.0)
    dinv_col = dinv.T
    dinv_ref[...] = dinv_col
    xs_ref[...] = x_ref[...] * dinv_col[:N]


def _prep(degp, x):
    return pl.pallas_call(
        _prep_body,
        out_shape=(
            jax.ShapeDtypeStruct((N, D), jnp.float32),
            jax.ShapeDtypeStruct((ACC_ROWS, 1), jnp.float32),
        ),
    )(degp, x)


_RB = 1000


def _final_body(aggp_ref, dinv_ref, out_ref):
    out_ref[...] = (aggp_ref[0] + aggp_ref[1]) * dinv_ref[...]


def _final(aggp, dinv):
    return pl.pallas_call(
        _final_body,
        grid=(N // _RB,),
        in_specs=[
            pl.BlockSpec((NC, _RB, D), lambda i: (0, i, 0)),
            pl.BlockSpec((_RB, 1), lambda i: (i, 0)),
        ],
        out_specs=pl.BlockSpec((_RB, D), lambda i: (i, 0)),
        out_shape=jax.ShapeDtypeStruct((N, D), jnp.float32),
    )(aggp, dinv)


def kernel(x, edge_index):
    ei = edge_index.astype(jnp.int32)
    pad = E_PAD - E
    frm2 = jnp.concatenate([ei[0], jnp.zeros((pad,), jnp.int32)]).reshape(
        E_PAD // CHUNK, CHUNK
    )
    to2d = jnp.concatenate([ei[1], jnp.full((pad,), N, jnp.int32)]).reshape(
        E_PAD // CHUNK, CHUNK
    )
    to3 = to2d.reshape(E_PAD // CHUNK, 1, CHUNK)
    fto = jnp.stack([frm2, to2d], axis=1)
    degp = _hist(to3)
    xs, dinv = _prep(degp, x)
    aggp = _agg(fto, xs)
    return _final(aggp, dinv)


# speedup vs baseline: 1.0321x; 1.0321x over previous
import dataclasses

import jax
import jax.numpy as jnp
from jax import lax
from jax.experimental import pallas as pl
from jax.experimental.pallas import tpu as pltpu
from jax.experimental.pallas import tpu_sc as plsc

N = 10000
D = 128
E = 320000
NC = 2
NS = 16
NW = NC * NS
CHUNK = 128
B_T = 80
EPT = B_T * CHUNK
E_PAD = EPT * NW
W0 = 148
ACC_ROWS = 10112
ACC_BLKS = ACC_ROWS // CHUNK
RPT = ACC_ROWS // NS

_MESH = plsc.VectorSubcoreMesh(
    core_axis_name="c", subcore_axis_name="s", num_cores=NC, num_subcores=NS
)

_CP = pltpu.CompilerParams()
if "needs_layout_passes" in pltpu.CompilerParams.__dataclass_fields__:
    _CP = dataclasses.replace(_CP, needs_layout_passes=False)


def _hist_body(to3_hbm, out_hbm, tbuf_v, cnt_v):
    c = lax.axis_index("c")
    s = lax.axis_index("s")
    wid = c * NS + s

    @pl.loop(0, ACC_ROWS // 16)
    def _(i):
        cnt_v[pl.ds(i * 16, 16)] = jnp.zeros((16,), jnp.float32)

    pltpu.sync_copy(to3_hbm.at[pl.ds(wid * B_T, B_T)], tbuf_v)

    @pl.loop(0, B_T)
    def _(b):
        @pl.loop(0, CHUNK // 16)
        def _(g):
            idx16 = tbuf_v[b, 0, pl.ds(g * 16, 16)]
            plsc.addupdate_scatter(cnt_v, [idx16], jnp.ones((16,), jnp.float32))

    pltpu.sync_copy(cnt_v, out_hbm.at[wid])


_hist = pl.kernel(
    _hist_body,
    out_type=jax.ShapeDtypeStruct((NW, ACC_ROWS), jnp.float32),
    mesh=_MESH,
    scratch_types=[
        pltpu.VMEM((B_T, 1, CHUNK), jnp.int32),
        pltpu.VMEM((ACC_ROWS,), jnp.float32),
    ],
    compiler_params=_CP,
)


def _agg_body(fto_hbm, xs_hbm, out_hbm,
              idx0_v, idx1_v, rows0_v, rows1_v, acc_sh,
              gsem0, gsem1, isem0, isem1, hsem0, hsem1):
    c = lax.axis_index("c")
    s = lax.axis_index("s")

    @pl.loop(0, CHUNK)
    def _(r):
        for g in range(D // 16):
            rows0_v[r, pl.ds(g * 16, 16)] = jnp.zeros((16,), jnp.float32)

    @pl.loop(0, (ACC_BLKS + NS - 1) // NS)
    def _(b):
        blk = b * NS + s

        @pl.when(blk < ACC_BLKS)
        def _():
            pltpu.sync_copy(rows0_v, acc_sh.at[pl.ds(blk * CHUNK, CHUNK)])

    nb = jnp.where(c == 0, W0, 2 * B_T - W0)
    base = jnp.where(c == 0, s * W0, NS * W0 + s * (2 * B_T - W0))
    pltpu.async_copy(fto_hbm.at[base], idx0_v, isem0)
    pltpu.async_copy(fto_hbm.at[base + 1], idx1_v, isem1)
    plsc.subcore_barrier()
    idxs = (idx0_v, idx1_v)
    rows = (rows0_v, rows1_v)
    gsems = (gsem0, gsem1)
    hsems = (hsem0, hsem1)
    isems = (isem0, isem1)

    HC = CHUNK // 2

    def _gather_start(j):
        pltpu.async_copy(
            xs_hbm.at[idxs[j].at[0].at[pl.ds(0, HC)]],
            rows[j].at[pl.ds(0, HC)], gsems[j])
        pltpu.async_copy(
            xs_hbm.at[idxs[j].at[0].at[pl.ds(HC, HC)]],
            rows[j].at[pl.ds(HC, HC)], hsems[j])

    def _gather_wait(j):
        pltpu.make_async_copy(
            xs_hbm.at[idxs[j].at[0].at[pl.ds(0, HC)]],
            rows[j].at[pl.ds(0, HC)], gsems[j]).wait()
        pltpu.make_async_copy(
            xs_hbm.at[idxs[j].at[0].at[pl.ds(HC, HC)]],
            rows[j].at[pl.ds(HC, HC)], hsems[j]).wait()

    pltpu.make_async_copy(fto_hbm.at[base], idx0_v, isem0).wait()
    _gather_start(0)

    @pl.loop(0, nb, step=2)
    def _(b):
        for j in range(2):
            bb = b + j
            j1 = 1 - j

            @pl.when(bb + 1 < nb)
            def _():
                pltpu.make_async_copy(fto_hbm.at[base], idxs[j1], isems[j1]).wait()
                _gather_start(j1)

            _gather_wait(j)
            pltpu.sync_copy(rows[j], acc_sh.at[idxs[j].at[1]], add=True)

            @pl.when(bb + 2 < nb)
            def _():
                pltpu.async_copy(fto_hbm.at[base + bb + 2], idxs[j], isems[j])

    plsc.subcore_barrier()
    pltpu.sync_copy(
        acc_sh.at[pl.ds(s * RPT, RPT)], out_hbm.at[c].at[pl.ds(s * RPT, RPT)]
    )


_agg = pl.kernel(
    _agg_body,
    out_type=jax.ShapeDtypeStruct((NC, ACC_ROWS, D), jnp.float32),
    mesh=_MESH,
    scratch_types=[
        pltpu.VMEM((2, CHUNK), jnp.int32),
        pltpu.VMEM((2, CHUNK), jnp.int32),
        pltpu.VMEM((CHUNK, D), jnp.float32),
        pltpu.VMEM((CHUNK, D), jnp.float32),
        pltpu.VMEM_SHARED((ACC_ROWS, D), jnp.float32),
        pltpu.SemaphoreType.DMA,
        pltpu.SemaphoreType.DMA,
        pltpu.SemaphoreType.DMA,
        pltpu.SemaphoreType.DMA,
        pltpu.SemaphoreType.DMA,
        pltpu.SemaphoreType.DMA,
    ],
)


def _prep_body(degp_ref, x_ref, xs_ref, dinv_ref):
    deg = jnp.sum(degp_ref[...], axis=0, keepdims=True)
    dinv = jnp.where(deg > 0.0, lax.rsqrt(deg), 0.0)
    dinv_col = dinv.T
    dinv_ref[...] = dinv_col
    xs_ref[...] = x_ref[...] * dinv_col[:N]


def _prep(degp, x):
    return pl.pallas_call(
        _prep_body,
        out_shape=(
            jax.ShapeDtypeStruct((N, D), jnp.float32),
            jax.ShapeDtypeStruct((ACC_ROWS, 1), jnp.float32),
        ),
    )(degp, x)


_RB = 1000


def _final_body(aggp_ref, dinv_ref, out_ref):
    out_ref[...] = (aggp_ref[0] + aggp_ref[1]) * dinv_ref[...]


def _final(aggp, dinv):
    return pl.pallas_call(
        _final_body,
        grid=(N // _RB,),
        in_specs=[
            pl.BlockSpec((NC, _RB, D), lambda i: (0, i, 0)),
            pl.BlockSpec((_RB, 1), lambda i: (i, 0)),
        ],
        out_specs=pl.BlockSpec((_RB, D), lambda i: (i, 0)),
        out_shape=jax.ShapeDtypeStruct((N, D), jnp.float32),
    )(aggp, dinv)


def kernel(x, edge_index):
    ei = edge_index.astype(jnp.int32)
    pad = E_PAD - E
    frm2 = jnp.concatenate([ei[0], jnp.zeros((pad,), jnp.int32)]).reshape(
        E_PAD // CHUNK, CHUNK
    )
    to2d = jnp.concatenate([ei[1], jnp.full((pad,), N, jnp.int32)]).reshape(
        E_PAD // CHUNK, CHUNK
    )
    to3 = to2d.reshape(E_PAD // CHUNK, 1, CHUNK)
    fto = jnp.stack([frm2, to2d], axis=1)
    degp = _hist(to3)
    xs, dinv = _prep(degp, x)
    aggp = _agg(fto, xs)
    return _final(aggp, dinv)
